# baseline (speedup 1.0000x reference)
import math

import jax
import jax.numpy as jnp
from jax import lax
from jax.experimental import pallas as pl
from jax.experimental.pallas import tpu as pltpu
from jax.experimental.pallas import tpu_sc as plsc

B, L, V, D, O = 4096, 200, 1000000, 64, 32
NC, NS = 2, 16
NW = NC * NS
BPW = B // NW
C0 = 128
C1 = L - C0
ROW_UNROLL = 8


NCH = 7812
NSLOT = 4
TAIL0 = NCH * 128


def _sc_transpose(tableT, tail128):
  mesh = plsc.VectorSubcoreMesh(core_axis_name="c", subcore_axis_name="s")

  def body(tab_hbm, tail_hbm, out_hbm, v0, v1, v2, v3, o0, o1, o2, o3,
           si0, si1, si2, si3, so0, so1, so2, so3):
    wid = lax.axis_index("s") * NC + lax.axis_index("c")
    vbufs = [v0, v1, v2, v3]
    obufs = [o0, o1, o2, o3]
    sis = [si0, si1, si2, si3]
    sos = [so0, so1, so2, so3]
    iota = lax.iota(jnp.int32, 16)
    iota64 = iota * D

    def chunk_of(k):
      return wid + NW * k

    def issue_in(ch, buf, sem):
      pltpu.async_copy(tab_hbm.at[:, pl.ds(ch * 128, 128)], buf, sem)

    def wait_in(buf, sem):
      pltpu.make_async_copy(tab_hbm.at[:, pl.ds(0, 128)], buf, sem).wait()

    def transpose(vbuf, obuf):
      def dstep(d, carry):
        for k in range(8):
          val = vbuf[d, pl.ds(16 * k, 16)]
          plsc.store_scatter(obuf, [iota64 + (k * 16 * D + d)], val)
        return carry

      lax.fori_loop(0, D, dstep, 0)

    def flush(ch, buf, sem):
      pltpu.async_copy(buf, out_hbm.at[pl.ds(ch * 128 * D, 128 * D)], sem)

    def wait_out(buf, sem):
      pltpu.make_async_copy(buf, out_hbm.at[pl.ds(0, 128 * D)], sem).wait()

    for s in range(NSLOT):
      @pl.when(chunk_of(s) < NCH)
      def _(s=s):
        issue_in(chunk_of(s), vbufs[s], sis[s])

    def step(p, carry):
      for s in range(NSLOT):
        k = p * NSLOT + s
        ch = chunk_of(k)

        @pl.when(ch < NCH)
        def _(s=s, k=k, ch=ch):
          wait_in(vbufs[s], sis[s])

          @pl.when(k >= NSLOT)
          def _():
            wait_out(obufs[s], sos[s])

          transpose(vbufs[s], obufs[s])
          flush(ch, obufs[s], sos[s])
          nxt = ch + NW * NSLOT

          @pl.when(nxt < NCH)
          def _():
            issue_in(nxt, vbufs[s], sis[s])

      return carry

    lax.fori_loop(0, (NCH // NW + NSLOT) // NSLOT + 1, step, 0)
    for s in range(NSLOT):
      @pl.when(chunk_of(s) < NCH)
      def _(s=s):
        wait_out(obufs[s], sos[s])

    @pl.when(wid == 0)
    def _():
      rem = V - TAIL0
      pltpu.sync_copy(tail_hbm, v0)

      def dstep(d, carry):
        for k in range(rem // 16):
          val = v0[d, pl.ds(16 * k, 16)]
          plsc.store_scatter(o0, [iota64 + (k * 16 * D + d)], val)
        return carry

      lax.fori_loop(0, D, dstep, 0)
      pltpu.sync_copy(o0.at[pl.ds(0, rem * D)],
                      out_hbm.at[pl.ds(TAIL0 * D, rem * D)])

  return pl.kernel(
      body,
      out_type=jax.ShapeDtypeStruct((V * D,), jnp.float32),
      mesh=mesh,
      scratch_types=(
          [pltpu.VMEM((D, 128), jnp.float32)] * NSLOT
          + [pltpu.VMEM((128 * D,), jnp.float32)] * NSLOT
          + [pltpu.SemaphoreType.DMA] * (2 * NSLOT)
      ),
      compiler_params=pltpu.CompilerParams(use_tc_tiling_on_sc=True,
                                           needs_layout_passes=False),
  )(tableT, tail128)


def _sc_gather_sum(tokens_flat, table):
  mesh = plsc.VectorSubcoreMesh(core_axis_name="c", subcore_axis_name="s")

  def body(tokens_hbm, table_hbm, out_hbm, idx_v, buf_a, buf_b, out_v,
           sem_a, sem_b):
    wid = lax.axis_index("s") * NC + lax.axis_index("c")
    base = wid * BPW
    pltpu.sync_copy(tokens_hbm.at[pl.ds(base * L, BPW * L)], idx_v)

    def issue(bag, buf, sem):
      off = bag * L
      pltpu.async_copy(table_hbm.at[idx_v.at[pl.ds(off, C0)]],
                       buf.at[pl.ds(0, C0)], sem)
      pltpu.async_copy(table_hbm.at[idx_v.at[pl.ds(off + C0, C1)]],
                       buf.at[pl.ds(C0, C1)], sem)

    def wait(buf, sem):
      pltpu.make_async_copy(table_hbm.at[pl.ds(0, L)], buf, sem).wait()

    def accumulate(bag, buf):
      zeros = jnp.zeros((16,), jnp.float32)

      def step(i, accs):
        r = i * ROW_UNROLL
        new = list(accs)
        for dr in range(ROW_UNROLL):
          for j in range(4):
            new[j] = new[j] + buf[r + dr, pl.ds(16 * j, 16)]
        return tuple(new)

      accs = lax.fori_loop(0, L // ROW_UNROLL, step,
                           (zeros, zeros, zeros, zeros))
      for j in range(4):
        out_v[pl.ds(bag * D + 16 * j, 16)] = accs[j]

    issue(0, buf_a, sem_a)

    def pair(p, carry):
      bag = p * 2
      issue(bag + 1, buf_b, sem_b)
      wait(buf_a, sem_a)
      accumulate(bag, buf_a)

      @pl.when(bag + 2 < BPW)
      def _():
        issue(bag + 2, buf_a, sem_a)

      wait(buf_b, sem_b)
      accumulate(bag + 1, buf_b)
      return carry

    lax.fori_loop(0, BPW // 2, pair, 0)
    pltpu.sync_copy(out_v, out_hbm.at[pl.ds(base * D, BPW * D)])

  return pl.kernel(
      body,
      out_type=jax.ShapeDtypeStruct((B * D,), jnp.float32),
      mesh=mesh,
      scratch_types=[
          pltpu.VMEM((BPW * L,), jnp.int32),
          pltpu.VMEM((L, D), jnp.float32),
          pltpu.VMEM((L, D), jnp.float32),
          pltpu.VMEM((BPW * D,), jnp.float32),
          pltpu.SemaphoreType.DMA,
          pltpu.SemaphoreType.DMA,
      ],
      compiler_params=pltpu.CompilerParams(use_tc_tiling_on_sc=False),
  )(tokens_flat, table)


def _tc_head(tokens, sums, W1, b1, W2, b2):
  def body(tok_ref, sums_ref, w1_ref, b1_ref, w2_ref, b2_ref, out_ref):
    t = tok_ref[...]
    cnt = jnp.sum((t != 0).astype(jnp.float32), axis=1, keepdims=True)
    pooled = sums_ref[...] / jnp.maximum(cnt, 1.0)
    h = jnp.dot(pooled, w1_ref[...],
                preferred_element_type=jnp.float32) + b1_ref[...]
    h = 0.5 * h * (1.0 + lax.erf(h * (1.0 / math.sqrt(2.0))))
    out_ref[...] = jnp.dot(h, w2_ref[...],
                           preferred_element_type=jnp.float32) + b2_ref[...]

  grid = 8
  bb = B // grid
  return pl.pallas_call(
      body,
      out_shape=jax.ShapeDtypeStruct((B, O), jnp.float32),
      grid=(grid,),
      in_specs=[
          pl.BlockSpec((bb, L), lambda i: (i, 0)),
          pl.BlockSpec((bb, D), lambda i: (i, 0)),
          pl.BlockSpec((D, D), lambda i: (0, 0)),
          pl.BlockSpec((1, D), lambda i: (0, 0)),
          pl.BlockSpec((D, O), lambda i: (0, 0)),
          pl.BlockSpec((1, O), lambda i: (0, 0)),
      ],
      out_specs=pl.BlockSpec((bb, O), lambda i: (i, 0)),
  )(tokens, sums, W1, b1, W2, b2)


def kernel(tokens, table, W1, b1, W2, b2):
  tokens = tokens.astype(jnp.int32)
  tableT = table.T
  tail128 = jnp.pad(tableT[:, TAIL0:], ((0, 0), (0, 128 - (V - TAIL0))))
  table_lin = _sc_transpose(tableT, tail128).reshape(V, D)
  sums = _sc_gather_sum(tokens.reshape(-1), table_lin).reshape(B, D)
  return _tc_head(tokens, sums, W1, b1.reshape(1, D), W2, b2.reshape(1, O))


# speedup vs baseline: 1.2256x; 1.2256x over previous
import math

import jax
import jax.numpy as jnp
from jax import lax
from jax.experimental import pallas as pl
from jax.experimental.pallas import tpu as pltpu
from jax.experimental.pallas import tpu_sc as plsc

B, L, V, D, O = 4096, 200, 1000000, 64, 32
NC, NS = 2, 16
NW = NC * NS
BPW = B // NW
C0 = 128
C1 = L - C0
ROW_UNROLL = 8


NCH = 7812
NSLOT = 4
TAIL0 = NCH * 128


def _sc_transpose(tableT, tail128):
  mesh = plsc.VectorSubcoreMesh(core_axis_name="c", subcore_axis_name="s")

  def body(tab_hbm, tail_hbm, out_hbm, v0, v1, v2, v3, p0, p1, p2, p3,
           o0, o1, o2, o3, si0, si1, si2, si3, so0, so1, so2, so3):
    wid = lax.axis_index("s") * NC + lax.axis_index("c")
    vbufs = [v0, v1, v2, v3]
    pbufs = [p0, p1, p2, p3]
    obufs = [o0, o1, o2, o3]
    sis = [si0, si1, si2, si3]
    sos = [so0, so1, so2, so3]
    iota = lax.iota(jnp.int32, 16)
    iota64 = iota * D

    def chunk_of(k):
      return wid + NW * k

    def issue_in(ch, buf, sem):
      pltpu.async_copy(tab_hbm.at[:, pl.ds(ch * 128, 128)], buf, sem)

    def wait_in(buf, sem):
      pltpu.make_async_copy(tab_hbm.at[:, pl.ds(0, 128)], buf, sem).wait()

    def transpose(vbuf, pbuf, obuf):
      def dstep(d, carry):
        for k in range(8):
          val = vbuf[d, pl.ds(16 * k, 16)]
          rot = (iota + (d + 16 * k)) & 63
          plsc.store_scatter(pbuf, [iota64 + 16 * k * D + rot], val)
        return carry

      lax.fori_loop(0, D, dstep, 0)

      def cstep(r, carry):
        for j in range(4):
          src = r * D + ((iota + (16 * j + r)) & 63)
          obuf[pl.ds(r * D + 16 * j, 16)] = plsc.load_gather(pbuf, [src])
        return carry

      lax.fori_loop(0, 128, cstep, 0)

    def flush(ch, buf, sem):
      pltpu.async_copy(buf, out_hbm.at[pl.ds(ch * 128 * D, 128 * D)], sem)

    def wait_out(buf, sem):
      pltpu.make_async_copy(buf, out_hbm.at[pl.ds(0, 128 * D)], sem).wait()

    for s in range(NSLOT):
      @pl.when(chunk_of(s) < NCH)
      def _(s=s):
        issue_in(chunk_of(s), vbufs[s], sis[s])

    def step(p, carry):
      for s in range(NSLOT):
        k = p * NSLOT + s
        ch = chunk_of(k)

        @pl.when(ch < NCH)
        def _(s=s, k=k, ch=ch):
          wait_in(vbufs[s], sis[s])

          @pl.when(k >= NSLOT)
          def _():
            wait_out(obufs[s], sos[s])

          transpose(vbufs[s], pbufs[s], obufs[s])
          flush(ch, obufs[s], sos[s])
          nxt = ch + NW * NSLOT

          @pl.when(nxt < NCH)
          def _():
            issue_in(nxt, vbufs[s], sis[s])

      return carry

    lax.fori_loop(0, (NCH // NW + NSLOT) // NSLOT + 1, step, 0)
    for s in range(NSLOT):
      @pl.when(chunk_of(s) < NCH)
      def _(s=s):
        wait_out(obufs[s], sos[s])

    @pl.when(wid == 0)
    def _():
      rem = V - TAIL0
      pltpu.sync_copy(tail_hbm, v0)

      def dstep(d, carry):
        for k in range(rem // 16):
          val = v0[d, pl.ds(16 * k, 16)]
          plsc.store_scatter(o0, [iota64 + (k * 16 * D + d)], val)
        return carry

      lax.fori_loop(0, D, dstep, 0)
      pltpu.sync_copy(o0.at[pl.ds(0, rem * D)],
                      out_hbm.at[pl.ds(TAIL0 * D, rem * D)])

  return pl.kernel(
      body,
      out_type=jax.ShapeDtypeStruct((V * D,), jnp.float32),
      mesh=mesh,
      scratch_types=(
          [pltpu.VMEM((D, 128), jnp.float32)] * NSLOT
          + [pltpu.VMEM((128 * D,), jnp.float32)] * (2 * NSLOT)
          + [pltpu.SemaphoreType.DMA] * (2 * NSLOT)
      ),
      compiler_params=pltpu.CompilerParams(use_tc_tiling_on_sc=True,
                                           needs_layout_passes=False),
  )(tableT, tail128)


def _sc_gather_sum(tokens_flat, table):
  mesh = plsc.VectorSubcoreMesh(core_axis_name="c", subcore_axis_name="s")

  def body(tokens_hbm, table_hbm, out_hbm, idx_v, buf_a, buf_b, out_v,
           sem_a, sem_b):
    wid = lax.axis_index("s") * NC + lax.axis_index("c")
    base = wid * BPW
    pltpu.sync_copy(tokens_hbm.at[pl.ds(base * L, BPW * L)], idx_v)

    def issue(bag, buf, sem):
      off = bag * L
      pltpu.async_copy(table_hbm.at[idx_v.at[pl.ds(off, C0)]],
                       buf.at[pl.ds(0, C0)], sem)
      pltpu.async_copy(table_hbm.at[idx_v.at[pl.ds(off + C0, C1)]],
                       buf.at[pl.ds(C0, C1)], sem)

    def wait(buf, sem):
      pltpu.make_async_copy(table_hbm.at[pl.ds(0, L)], buf, sem).wait()

    def accumulate(bag, buf):
      zeros = jnp.zeros((16,), jnp.float32)

      def step(i, accs):
        r = i * ROW_UNROLL
        new = list(accs)
        for dr in range(ROW_UNROLL):
          for j in range(4):
            new[j] = new[j] + buf[r + dr, pl.ds(16 * j, 16)]
        return tuple(new)

      accs = lax.fori_loop(0, L // ROW_UNROLL, step,
                           (zeros, zeros, zeros, zeros))
      for j in range(4):
        out_v[pl.ds(bag * D + 16 * j, 16)] = accs[j]

    issue(0, buf_a, sem_a)

    def pair(p, carry):
      bag = p * 2
      issue(bag + 1, buf_b, sem_b)
      wait(buf_a, sem_a)
      accumulate(bag, buf_a)

      @pl.when(bag + 2 < BPW)
      def _():
        issue(bag + 2, buf_a, sem_a)

      wait(buf_b, sem_b)
      accumulate(bag + 1, buf_b)
      return carry

    lax.fori_loop(0, BPW // 2, pair, 0)
    pltpu.sync_copy(out_v, out_hbm.at[pl.ds(base * D, BPW * D)])

  return pl.kernel(
      body,
      out_type=jax.ShapeDtypeStruct((B * D,), jnp.float32),
      mesh=mesh,
      scratch_types=[
          pltpu.VMEM((BPW * L,), jnp.int32),
          pltpu.VMEM((L, D), jnp.float32),
          pltpu.VMEM((L, D), jnp.float32),
          pltpu.VMEM((BPW * D,), jnp.float32),
          pltpu.SemaphoreType.DMA,
          pltpu.SemaphoreType.DMA,
      ],
      compiler_params=pltpu.CompilerParams(use_tc_tiling_on_sc=False),
  )(tokens_flat, table)


def _tc_head(tokens, sums, W1, b1, W2, b2):
  def body(tok_ref, sums_ref, w1_ref, b1_ref, w2_ref, b2_ref, out_ref):
    t = tok_ref[...]
    cnt = jnp.sum((t != 0).astype(jnp.float32), axis=1, keepdims=True)
    pooled = sums_ref[...] / jnp.maximum(cnt, 1.0)
    h = jnp.dot(pooled, w1_ref[...],
                preferred_element_type=jnp.float32) + b1_ref[...]
    h = 0.5 * h * (1.0 + lax.erf(h * (1.0 / math.sqrt(2.0))))
    out_ref[...] = jnp.dot(h, w2_ref[...],
                           preferred_element_type=jnp.float32) + b2_ref[...]

  grid = 8
  bb = B // grid
  return pl.pallas_call(
      body,
      out_shape=jax.ShapeDtypeStruct((B, O), jnp.float32),
      grid=(grid,),
      in_specs=[
          pl.BlockSpec((bb, L), lambda i: (i, 0)),
          pl.BlockSpec((bb, D), lambda i: (i, 0)),
          pl.BlockSpec((D, D), lambda i: (0, 0)),
          pl.BlockSpec((1, D), lambda i: (0, 0)),
          pl.BlockSpec((D, O), lambda i: (0, 0)),
          pl.BlockSpec((1, O), lambda i: (0, 0)),
      ],
      out_specs=pl.BlockSpec((bb, O), lambda i: (i, 0)),
  )(tokens, sums, W1, b1, W2, b2)


def kernel(tokens, table, W1, b1, W2, b2):
  tokens = tokens.astype(jnp.int32)
  tableT = table.T
  tail128 = jnp.pad(tableT[:, TAIL0:], ((0, 0), (0, 128 - (V - TAIL0))))
  table_lin = _sc_transpose(tableT, tail128).reshape(V, D)
  sums = _sc_gather_sum(tokens.reshape(-1), table_lin).reshape(B, D)
  return _tc_head(tokens, sums, W1, b1.reshape(1, D), W2, b2.reshape(1, O))


# speedup vs baseline: 1.2832x; 1.0471x over previous
import math

import jax
import jax.numpy as jnp
from jax import lax
from jax.experimental import pallas as pl
from jax.experimental.pallas import tpu as pltpu
from jax.experimental.pallas import tpu_sc as plsc

B, L, V, D, O = 4096, 200, 1000000, 64, 32
NC, NS = 2, 16
NW = NC * NS
BPW = B // NW
C0 = 128
C1 = L - C0
ROW_UNROLL = 8


NCH = 7812
NSLOT = 4
TAIL0 = NCH * 128


def _sc_transpose(tableT, tail128):
  mesh = plsc.VectorSubcoreMesh(core_axis_name="c", subcore_axis_name="s")

  def body(tab_hbm, tail_hbm, out_hbm, v0, v1, v2, v3, p0, p1, p2, p3,
           o0, o1, o2, o3, si0, si1, si2, si3, so0, so1, so2, so3):
    wid = lax.axis_index("s") * NC + lax.axis_index("c")
    vbufs = [v0, v1, v2, v3]
    pbufs = [p0, p1, p2, p3]
    obufs = [o0, o1, o2, o3]
    sis = [si0, si1, si2, si3]
    sos = [so0, so1, so2, so3]
    iota = lax.iota(jnp.int32, 16)
    iota64 = iota * D

    def chunk_of(k):
      return wid + NW * k

    def issue_in(ch, buf, sem):
      pltpu.async_copy(tab_hbm.at[:, pl.ds(ch * 128, 128)], buf, sem)

    def wait_in(buf, sem):
      pltpu.make_async_copy(tab_hbm.at[:, pl.ds(0, 128)], buf, sem).wait()

    def transpose(vbuf, pbuf, obuf):
      def dstep(d, carry):
        for k in range(8):
          val = vbuf[d, pl.ds(16 * k, 16)]
          rot = (iota + (d + 16 * k)) & 63
          plsc.store_scatter(pbuf, [iota64 + 16 * k * D + rot], val)
        return carry

      lax.fori_loop(0, D, dstep, 0)

      def cstep(r, carry):
        for j in range(4):
          val = pbuf[pl.ds(r * D + 16 * j, 16)]
          dst = r * D + ((iota + (16 * j - r)) & 63)
          plsc.store_scatter(obuf, [dst], val)
        return carry

      lax.fori_loop(0, 128, cstep, 0)

    def flush(ch, buf, sem):
      pltpu.async_copy(buf, out_hbm.at[pl.ds(ch * 128 * D, 128 * D)], sem)

    def wait_out(buf, sem):
      pltpu.make_async_copy(buf, out_hbm.at[pl.ds(0, 128 * D)], sem).wait()

    for s in range(NSLOT):
      @pl.when(chunk_of(s) < NCH)
      def _(s=s):
        issue_in(chunk_of(s), vbufs[s], sis[s])

    def step(p, carry):
      for s in range(NSLOT):
        k = p * NSLOT + s
        ch = chunk_of(k)

        @pl.when(ch < NCH)
        def _(s=s, k=k, ch=ch):
          wait_in(vbufs[s], sis[s])

          @pl.when(k >= NSLOT)
          def _():
            wait_out(obufs[s], sos[s])

          transpose(vbufs[s], pbufs[s], obufs[s])
          flush(ch, obufs[s], sos[s])
          nxt = ch + NW * NSLOT

          @pl.when(nxt < NCH)
          def _():
            issue_in(nxt, vbufs[s], sis[s])

      return carry

    lax.fori_loop(0, (NCH // NW + NSLOT) // NSLOT + 1, step, 0)
    for s in range(NSLOT):
      @pl.when(chunk_of(s) < NCH)
      def _(s=s):
        wait_out(obufs[s], sos[s])

    @pl.when(wid == 0)
    def _():
      rem = V - TAIL0
      pltpu.sync_copy(tail_hbm, v0)

      def dstep(d, carry):
        for k in range(rem // 16):
          val = v0[d, pl.ds(16 * k, 16)]
          plsc.store_scatter(o0, [iota64 + (k * 16 * D + d)], val)
        return carry

      lax.fori_loop(0, D, dstep, 0)
      pltpu.sync_copy(o0.at[pl.ds(0, rem * D)],
                      out_hbm.at[pl.ds(TAIL0 * D, rem * D)])

  return pl.kernel(
      body,
      out_type=jax.ShapeDtypeStruct((V * D,), jnp.float32),
      mesh=mesh,
      scratch_types=(
          [pltpu.VMEM((D, 128), jnp.float32)] * NSLOT
          + [pltpu.VMEM((128 * D,), jnp.float32)] * (2 * NSLOT)
          + [pltpu.SemaphoreType.DMA] * (2 * NSLOT)
      ),
      compiler_params=pltpu.CompilerParams(use_tc_tiling_on_sc=True,
                                           needs_layout_passes=False),
  )(tableT, tail128)


def _sc_gather_sum(tokens_flat, table):
  mesh = plsc.VectorSubcoreMesh(core_axis_name="c", subcore_axis_name="s")

  def body(tokens_hbm, table_hbm, out_hbm, idx_v, buf_a, buf_b, out_v,
           sem_a, sem_b):
    wid = lax.axis_index("s") * NC + lax.axis_index("c")
    base = wid * BPW
    pltpu.sync_copy(tokens_hbm.at[pl.ds(base * L, BPW * L)], idx_v)

    def issue(bag, buf, sem):
      off = bag * L
      pltpu.async_copy(table_hbm.at[idx_v.at[pl.ds(off, C0)]],
                       buf.at[pl.ds(0, C0)], sem)
      pltpu.async_copy(table_hbm.at[idx_v.at[pl.ds(off + C0, C1)]],
                       buf.at[pl.ds(C0, C1)], sem)

    def wait(buf, sem):
      pltpu.make_async_copy(table_hbm.at[pl.ds(0, L)], buf, sem).wait()

    def accumulate(bag, buf):
      zeros = jnp.zeros((16,), jnp.float32)

      def step(i, accs):
        r = i * ROW_UNROLL
        new = list(accs)
        for dr in range(ROW_UNROLL):
          for j in range(4):
            new[j] = new[j] + buf[r + dr, pl.ds(16 * j, 16)]
        return tuple(new)

      accs = lax.fori_loop(0, L // ROW_UNROLL, step,
                           (zeros, zeros, zeros, zeros))
      for j in range(4):
        out_v[pl.ds(bag * D + 16 * j, 16)] = accs[j]

    issue(0, buf_a, sem_a)

    def pair(p, carry):
      bag = p * 2
      issue(bag + 1, buf_b, sem_b)
      wait(buf_a, sem_a)
      accumulate(bag, buf_a)

      @pl.when(bag + 2 < BPW)
      def _():
        issue(bag + 2, buf_a, sem_a)

      wait(buf_b, sem_b)
      accumulate(bag + 1, buf_b)
      return carry

    lax.fori_loop(0, BPW // 2, pair, 0)
    pltpu.sync_copy(out_v, out_hbm.at[pl.ds(base * D, BPW * D)])

  return pl.kernel(
      body,
      out_type=jax.ShapeDtypeStruct((B * D,), jnp.float32),
      mesh=mesh,
      scratch_types=[
          pltpu.VMEM((BPW * L,), jnp.int32),
          pltpu.VMEM((L, D), jnp.float32),
          pltpu.VMEM((L, D), jnp.float32),
          pltpu.VMEM((BPW * D,), jnp.float32),
          pltpu.SemaphoreType.DMA,
          pltpu.SemaphoreType.DMA,
      ],
      compiler_params=pltpu.CompilerParams(use_tc_tiling_on_sc=False),
  )(tokens_flat, table)


def _tc_head(tokens, sums, W1, b1, W2, b2):
  def body(tok_ref, sums_ref, w1_ref, b1_ref, w2_ref, b2_ref, out_ref):
    t = tok_ref[...]
    cnt = jnp.sum((t != 0).astype(jnp.float32), axis=1, keepdims=True)
    pooled = sums_ref[...] / jnp.maximum(cnt, 1.0)
    h = jnp.dot(pooled, w1_ref[...],
                preferred_element_type=jnp.float32) + b1_ref[...]
    h = 0.5 * h * (1.0 + lax.erf(h * (1.0 / math.sqrt(2.0))))
    out_ref[...] = jnp.dot(h, w2_ref[...],
                           preferred_element_type=jnp.float32) + b2_ref[...]

  grid = 8
  bb = B // grid
  return pl.pallas_call(
      body,
      out_shape=jax.ShapeDtypeStruct((B, O), jnp.float32),
      grid=(grid,),
      in_specs=[
          pl.BlockSpec((bb, L), lambda i: (i, 0)),
          pl.BlockSpec((bb, D), lambda i: (i, 0)),
          pl.BlockSpec((D, D), lambda i: (0, 0)),
          pl.BlockSpec((1, D), lambda i: (0, 0)),
          pl.BlockSpec((D, O), lambda i: (0, 0)),
          pl.BlockSpec((1, O), lambda i: (0, 0)),
      ],
      out_specs=pl.BlockSpec((bb, O), lambda i: (i, 0)),
  )(tokens, sums, W1, b1, W2, b2)


def kernel(tokens, table, W1, b1, W2, b2):
  tokens = tokens.astype(jnp.int32)
  tableT = table.T
  tail128 = jnp.pad(tableT[:, TAIL0:], ((0, 0), (0, 128 - (V - TAIL0))))
  table_lin = _sc_transpose(tableT, tail128).reshape(V, D)
  sums = _sc_gather_sum(tokens.reshape(-1), table_lin).reshape(B, D)
  return _tc_head(tokens, sums, W1, b1.reshape(1, D), W2, b2.reshape(1, O))


# speedup vs baseline: 1.6727x; 1.3035x over previous
import math

import jax
import jax.numpy as jnp
from jax import lax
from jax.experimental import pallas as pl
from jax.experimental.pallas import tpu as pltpu
from jax.experimental.pallas import tpu_sc as plsc

B, L, V, D, O = 4096, 200, 1000000, 64, 32
NC, NS = 2, 16
NW = NC * NS
BPW = B // NW
C0 = 128
C1 = L - C0
ROW_UNROLL = 8


NCH = 7812
NSLOT = 4
TAIL0 = NCH * 128


def _sc_transpose(tableT, tail128):
  mesh = plsc.VectorSubcoreMesh(core_axis_name="c", subcore_axis_name="s")

  def body(tab_hbm, tail_hbm, out_hbm, v0, v1, v2, v3,
           o0, o1, o2, o3, si0, si1, si2, si3, so0, so1, so2, so3):
    wid = lax.axis_index("s") * NC + lax.axis_index("c")
    vbufs = [v0, v1, v2, v3]
    obufs = [o0, o1, o2, o3]
    sis = [si0, si1, si2, si3]
    sos = [so0, so1, so2, so3]
    iota = lax.iota(jnp.int32, 16)
    iota64 = iota * D

    def chunk_of(k):
      return wid + NW * k

    def issue_in(ch, buf, sem):
      pltpu.async_copy(tab_hbm.at[:, pl.ds(ch * 128, 128)], buf, sem)

    def wait_in(buf, sem):
      pltpu.make_async_copy(tab_hbm.at[:, pl.ds(0, 128)], buf, sem).wait()

    diag = [(iota + dlt) & 15 for dlt in range(16)]

    def transpose(vbuf, obuf):
      def rstep(rr, carry):
        r_idx = iota + rr * 16
        dst_r = iota * D + rr * 16 * D
        for d0 in (0, 16, 32, 48):
          for dlt in range(16):
            d_idx = diag[dlt] + d0
            val = plsc.load_gather(vbuf, [d_idx, r_idx])
            plsc.store_scatter(obuf, [dst_r + d_idx], val)
        return carry

      lax.fori_loop(0, 8, rstep, 0)

    def flush(ch, buf, sem):
      pltpu.async_copy(buf, out_hbm.at[pl.ds(ch * 128 * D, 128 * D)], sem)

    def wait_out(buf, sem):
      pltpu.make_async_copy(buf, out_hbm.at[pl.ds(0, 128 * D)], sem).wait()

    for s in range(NSLOT):
      @pl.when(chunk_of(s) < NCH)
      def _(s=s):
        issue_in(chunk_of(s), vbufs[s], sis[s])

    def step(p, carry):
      for s in range(NSLOT):
        k = p * NSLOT + s
        ch = chunk_of(k)

        @pl.when(ch < NCH)
        def _(s=s, k=k, ch=ch):
          wait_in(vbufs[s], sis[s])

          @pl.when(k >= NSLOT)
          def _():
            wait_out(obufs[s], sos[s])

          transpose(vbufs[s], obufs[s])
          flush(ch, obufs[s], sos[s])
          nxt = ch + NW * NSLOT

          @pl.when(nxt < NCH)
          def _():
            issue_in(nxt, vbufs[s], sis[s])

      return carry

    lax.fori_loop(0, (NCH // NW + NSLOT) // NSLOT + 1, step, 0)
    for s in range(NSLOT):
      @pl.when(chunk_of(s) < NCH)
      def _(s=s):
        wait_out(obufs[s], sos[s])

    @pl.when(wid == 0)
    def _():
      rem = V - TAIL0
      pltpu.sync_copy(tail_hbm, v0)

      def dstep(d, carry):
        for k in range(rem // 16):
          val = v0[d, pl.ds(16 * k, 16)]
          plsc.store_scatter(o0, [iota64 + (k * 16 * D + d)], val)
        return carry

      lax.fori_loop(0, D, dstep, 0)
      pltpu.sync_copy(o0.at[pl.ds(0, rem * D)],
                      out_hbm.at[pl.ds(TAIL0 * D, rem * D)])

  return pl.kernel(
      body,
      out_type=jax.ShapeDtypeStruct((V * D,), jnp.float32),
      mesh=mesh,
      scratch_types=(
          [pltpu.VMEM((D, 128), jnp.float32)] * NSLOT
          + [pltpu.VMEM((128 * D,), jnp.float32)] * NSLOT
          + [pltpu.SemaphoreType.DMA] * (2 * NSLOT)
      ),
      compiler_params=pltpu.CompilerParams(use_tc_tiling_on_sc=True,
                                           needs_layout_passes=False),
  )(tableT, tail128)


def _sc_gather_sum(tokens_flat, table):
  mesh = plsc.VectorSubcoreMesh(core_axis_name="c", subcore_axis_name="s")

  def body(tokens_hbm, table_hbm, out_hbm, idx_v, buf_a, buf_b, out_v,
           sem_a, sem_b):
    wid = lax.axis_index("s") * NC + lax.axis_index("c")
    base = wid * BPW
    pltpu.sync_copy(tokens_hbm.at[pl.ds(base * L, BPW * L)], idx_v)

    def issue(bag, buf, sem):
      off = bag * L
      pltpu.async_copy(table_hbm.at[idx_v.at[pl.ds(off, C0)]],
                       buf.at[pl.ds(0, C0)], sem)
      pltpu.async_copy(table_hbm.at[idx_v.at[pl.ds(off + C0, C1)]],
                       buf.at[pl.ds(C0, C1)], sem)

    def wait(buf, sem):
      pltpu.make_async_copy(table_hbm.at[pl.ds(0, L)], buf, sem).wait()

    def accumulate(bag, buf):
      zeros = jnp.zeros((16,), jnp.float32)

      def step(i, accs):
        r = i * ROW_UNROLL
        new = list(accs)
        for dr in range(ROW_UNROLL):
          for j in range(4):
            new[j] = new[j] + buf[r + dr, pl.ds(16 * j, 16)]
        return tuple(new)

      accs = lax.fori_loop(0, L // ROW_UNROLL, step,
                           (zeros, zeros, zeros, zeros))
      for j in range(4):
        out_v[pl.ds(bag * D + 16 * j, 16)] = accs[j]

    issue(0, buf_a, sem_a)

    def pair(p, carry):
      bag = p * 2
      issue(bag + 1, buf_b, sem_b)
      wait(buf_a, sem_a)
      accumulate(bag, buf_a)

      @pl.when(bag + 2 < BPW)
      def _():
        issue(bag + 2, buf_a, sem_a)

      wait(buf_b, sem_b)
      accumulate(bag + 1, buf_b)
      return carry

    lax.fori_loop(0, BPW // 2, pair, 0)
    pltpu.sync_copy(out_v, out_hbm.at[pl.ds(base * D, BPW * D)])

  return pl.kernel(
      body,
      out_type=jax.ShapeDtypeStruct((B * D,), jnp.float32),
      mesh=mesh,
      scratch_types=[
          pltpu.VMEM((BPW * L,), jnp.int32),
          pltpu.VMEM((L, D), jnp.float32),
          pltpu.VMEM((L, D), jnp.float32),
          pltpu.VMEM((BPW * D,), jnp.float32),
          pltpu.SemaphoreType.DMA,
          pltpu.SemaphoreType.DMA,
      ],
      compiler_params=pltpu.CompilerParams(use_tc_tiling_on_sc=False),
  )(tokens_flat, table)


def _tc_head(tokens, sums, W1, b1, W2, b2):
  def body(tok_ref, sums_ref, w1_ref, b1_ref, w2_ref, b2_ref, out_ref):
    t = tok_ref[...]
    cnt = jnp.sum((t != 0).astype(jnp.float32), axis=1, keepdims=True)
    pooled = sums_ref[...] / jnp.maximum(cnt, 1.0)
    h = jnp.dot(pooled, w1_ref[...],
                preferred_element_type=jnp.float32) + b1_ref[...]
    h = 0.5 * h * (1.0 + lax.erf(h * (1.0 / math.sqrt(2.0))))
    out_ref[...] = jnp.dot(h, w2_ref[...],
                           preferred_element_type=jnp.float32) + b2_ref[...]

  grid = 8
  bb = B // grid
  return pl.pallas_call(
      body,
      out_shape=jax.ShapeDtypeStruct((B, O), jnp.float32),
      grid=(grid,),
      in_specs=[
          pl.BlockSpec((bb, L), lambda i: (i, 0)),
          pl.BlockSpec((bb, D), lambda i: (i, 0)),
          pl.BlockSpec((D, D), lambda i: (0, 0)),
          pl.BlockSpec((1, D), lambda i: (0, 0)),
          pl.BlockSpec((D, O), lambda i: (0, 0)),
          pl.BlockSpec((1, O), lambda i: (0, 0)),
      ],
      out_specs=pl.BlockSpec((bb, O), lambda i: (i, 0)),
  )(tokens, sums, W1, b1, W2, b2)


def kernel(tokens, table, W1, b1, W2, b2):
  tokens = tokens.astype(jnp.int32)
  tableT = table.T
  tail128 = jnp.pad(tableT[:, TAIL0:], ((0, 0), (0, 128 - (V - TAIL0))))
  table_lin = _sc_transpose(tableT, tail128).reshape(V, D)
  sums = _sc_gather_sum(tokens.reshape(-1), table_lin).reshape(B, D)
  return _tc_head(tokens, sums, W1, b1.reshape(1, D), W2, b2.reshape(1, O))


# speedup vs baseline: 2.8337x; 1.6941x over previous
import math

import jax
import jax.numpy as jnp
from jax import lax
from jax.experimental import pallas as pl
from jax.experimental.pallas import tpu as pltpu
from jax.experimental.pallas import tpu_sc as plsc

B, L, V, D, O = 4096, 200, 1000000, 64, 32
NC, NS = 2, 16
NW = NC * NS
BPW = B // NW
C0 = 128
C1 = L - C0
ROW_UNROLL = 8


NCH = 7812
NSLOT = 4
TAIL0 = NCH * 128


def _sc_transpose(tableT, tail128):
  mesh = plsc.VectorSubcoreMesh(core_axis_name="c", subcore_axis_name="s")

  def body(tab_hbm, tail_hbm, out_hbm, v0, v1, v2, v3,
           o0, o1, o2, o3, si0, si1, si2, si3, so0, so1, so2, so3):
    wid = lax.axis_index("s") * NC + lax.axis_index("c")
    vbufs = [v0, v1, v2, v3]
    obufs = [o0, o1, o2, o3]
    sis = [si0, si1, si2, si3]
    sos = [so0, so1, so2, so3]
    iota = lax.iota(jnp.int32, 16)
    iota64 = iota * D

    def chunk_of(k):
      return wid + NW * k

    def issue_in(ch, buf, sem):
      pltpu.async_copy(tab_hbm.at[:, pl.ds(ch * 128, 128)], buf, sem)

    def wait_in(buf, sem):
      pltpu.make_async_copy(tab_hbm.at[:, pl.ds(0, 128)], buf, sem).wait()

    diag = [(iota + dlt) & 15 for dlt in range(16)]

    def transpose(vbuf, obuf):
      def rstep(rr, carry):
        r_idx = iota + rr * 16
        dst_r = iota * D + rr * 16 * D
        for d0 in (0, 16, 32, 48):
          for g in (0, 8):
            vals = []
            for dlt in range(g, g + 8):
              d_idx = diag[dlt] + d0
              vals.append(plsc.load_gather(vbuf, [d_idx, r_idx]))
            for i, dlt in enumerate(range(g, g + 8)):
              plsc.store_scatter(obuf, [dst_r + diag[dlt] + d0], vals[i])
        return carry

      lax.fori_loop(0, 8, rstep, 0)

    def flush(ch, buf, sem):
      pltpu.async_copy(buf, out_hbm.at[pl.ds(ch * 128 * D, 128 * D)], sem)

    def wait_out(buf, sem):
      pltpu.make_async_copy(buf, out_hbm.at[pl.ds(0, 128 * D)], sem).wait()

    for s in range(NSLOT):
      @pl.when(chunk_of(s) < NCH)
      def _(s=s):
        issue_in(chunk_of(s), vbufs[s], sis[s])

    def step(p, carry):
      for s in range(NSLOT):
        k = p * NSLOT + s
        ch = chunk_of(k)

        @pl.when(ch < NCH)
        def _(s=s, k=k, ch=ch):
          wait_in(vbufs[s], sis[s])

          @pl.when(k >= NSLOT)
          def _():
            wait_out(obufs[s], sos[s])

          transpose(vbufs[s], obufs[s])
          flush(ch, obufs[s], sos[s])
          nxt = ch + NW * NSLOT

          @pl.when(nxt < NCH)
          def _():
            issue_in(nxt, vbufs[s], sis[s])

      return carry

    lax.fori_loop(0, (NCH // NW + NSLOT) // NSLOT + 1, step, 0)
    for s in range(NSLOT):
      @pl.when(chunk_of(s) < NCH)
      def _(s=s):
        wait_out(obufs[s], sos[s])

    @pl.when(wid == 0)
    def _():
      rem = V - TAIL0
      pltpu.sync_copy(tail_hbm, v0)

      def dstep(d, carry):
        for k in range(rem // 16):
          val = v0[d, pl.ds(16 * k, 16)]
          plsc.store_scatter(o0, [iota64 + (k * 16 * D + d)], val)
        return carry

      lax.fori_loop(0, D, dstep, 0)
      pltpu.sync_copy(o0.at[pl.ds(0, rem * D)],
                      out_hbm.at[pl.ds(TAIL0 * D, rem * D)])

  return pl.kernel(
      body,
      out_type=jax.ShapeDtypeStruct((V * D,), jnp.float32),
      mesh=mesh,
      scratch_types=(
          [pltpu.VMEM((D, 128), jnp.float32)] * NSLOT
          + [pltpu.VMEM((128 * D,), jnp.float32)] * NSLOT
          + [pltpu.SemaphoreType.DMA] * (2 * NSLOT)
      ),
      compiler_params=pltpu.CompilerParams(use_tc_tiling_on_sc=True,
                                           needs_layout_passes=False),
  )(tableT, tail128)


def _sc_gather_sum(tokens_flat, table):
  mesh = plsc.VectorSubcoreMesh(core_axis_name="c", subcore_axis_name="s")

  def body(tokens_hbm, table_hbm, out_hbm, idx_v, buf_a, buf_b, out_v,
           sem_a, sem_b):
    wid = lax.axis_index("s") * NC + lax.axis_index("c")
    base = wid * BPW
    pltpu.sync_copy(tokens_hbm.at[pl.ds(base * L, BPW * L)], idx_v)

    def issue(bag, buf, sem):
      off = bag * L
      pltpu.async_copy(table_hbm.at[idx_v.at[pl.ds(off, C0)]],
                       buf.at[pl.ds(0, C0)], sem)
      pltpu.async_copy(table_hbm.at[idx_v.at[pl.ds(off + C0, C1)]],
                       buf.at[pl.ds(C0, C1)], sem)

    def wait(buf, sem):
      pltpu.make_async_copy(table_hbm.at[pl.ds(0, L)], buf, sem).wait()

    def accumulate(bag, buf):
      zeros = jnp.zeros((16,), jnp.float32)

      def step(i, accs):
        r = i * ROW_UNROLL
        new = list(accs)
        for dr in range(ROW_UNROLL):
          for j in range(4):
            new[j] = new[j] + buf[r + dr, pl.ds(16 * j, 16)]
        return tuple(new)

      accs = lax.fori_loop(0, L // ROW_UNROLL, step,
                           (zeros, zeros, zeros, zeros))
      for j in range(4):
        out_v[pl.ds(bag * D + 16 * j, 16)] = accs[j]

    issue(0, buf_a, sem_a)

    def pair(p, carry):
      bag = p * 2
      issue(bag + 1, buf_b, sem_b)
      wait(buf_a, sem_a)
      accumulate(bag, buf_a)

      @pl.when(bag + 2 < BPW)
      def _():
        issue(bag + 2, buf_a, sem_a)

      wait(buf_b, sem_b)
      accumulate(bag + 1, buf_b)
      return carry

    lax.fori_loop(0, BPW // 2, pair, 0)
    pltpu.sync_copy(out_v, out_hbm.at[pl.ds(base * D, BPW * D)])

  return pl.kernel(
      body,
      out_type=jax.ShapeDtypeStruct((B * D,), jnp.float32),
      mesh=mesh,
      scratch_types=[
          pltpu.VMEM((BPW * L,), jnp.int32),
          pltpu.VMEM((L, D), jnp.float32),
          pltpu.VMEM((L, D), jnp.float32),
          pltpu.VMEM((BPW * D,), jnp.float32),
          pltpu.SemaphoreType.DMA,
          pltpu.SemaphoreType.DMA,
      ],
      compiler_params=pltpu.CompilerParams(use_tc_tiling_on_sc=False),
  )(tokens_flat, table)


def _tc_head(tokens, sums, W1, b1, W2, b2):
  def body(tok_ref, sums_ref, w1_ref, b1_ref, w2_ref, b2_ref, out_ref):
    t = tok_ref[...]
    cnt = jnp.sum((t != 0).astype(jnp.float32), axis=1, keepdims=True)
    pooled = sums_ref[...] / jnp.maximum(cnt, 1.0)
    h = jnp.dot(pooled, w1_ref[...],
                preferred_element_type=jnp.float32) + b1_ref[...]
    h = 0.5 * h * (1.0 + lax.erf(h * (1.0 / math.sqrt(2.0))))
    out_ref[...] = jnp.dot(h, w2_ref[...],
                           preferred_element_type=jnp.float32) + b2_ref[...]

  grid = 8
  bb = B // grid
  return pl.pallas_call(
      body,
      out_shape=jax.ShapeDtypeStruct((B, O), jnp.float32),
      grid=(grid,),
      in_specs=[
          pl.BlockSpec((bb, L), lambda i: (i, 0)),
          pl.BlockSpec((bb, D), lambda i: (i, 0)),
          pl.BlockSpec((D, D), lambda i: (0, 0)),
          pl.BlockSpec((1, D), lambda i: (0, 0)),
          pl.BlockSpec((D, O), lambda i: (0, 0)),
          pl.BlockSpec((1, O), lambda i: (0, 0)),
      ],
      out_specs=pl.BlockSpec((bb, O), lambda i: (i, 0)),
  )(tokens, sums, W1, b1, W2, b2)


def kernel(tokens, table, W1, b1, W2, b2):
  tokens = tokens.astype(jnp.int32)
  tableT = table.T
  tail128 = jnp.pad(tableT[:, TAIL0:], ((0, 0), (0, 128 - (V - TAIL0))))
  table_lin = _sc_transpose(tableT, tail128).reshape(V, D)
  sums = _sc_gather_sum(tokens.reshape(-1), table_lin).reshape(B, D)
  return _tc_head(tokens, sums, W1, b1.reshape(1, D), W2, b2.reshape(1, O))


# speedup vs baseline: 3.0691x; 1.0831x over previous
import math

import jax
import jax.numpy as jnp
from jax import lax
from jax.experimental import pallas as pl
from jax.experimental.pallas import tpu as pltpu
from jax.experimental.pallas import tpu_sc as plsc

B, L, V, D, O = 4096, 200, 1000000, 64, 32
NC, NS = 2, 16
NW = NC * NS
BPW = B // NW
C0 = 128
C1 = L - C0
ROW_UNROLL = 8


NCH = 7812
NSLOT = 4
TAIL0 = NCH * 128


def _sc_transpose(tableT, tail128):
  mesh = plsc.VectorSubcoreMesh(core_axis_name="c", subcore_axis_name="s")

  def body(tab_hbm, tail_hbm, out_hbm, v0, v1, v2, v3,
           o0, o1, o2, o3, si0, si1, si2, si3, so0, so1, so2, so3):
    wid = lax.axis_index("s") * NC + lax.axis_index("c")
    vbufs = [v0, v1, v2, v3]
    obufs = [o0, o1, o2, o3]
    sis = [si0, si1, si2, si3]
    sos = [so0, so1, so2, so3]
    iota = lax.iota(jnp.int32, 16)
    iota64 = iota * D

    def chunk_of(k):
      return wid + NW * k

    def issue_in(ch, buf, sem):
      pltpu.async_copy(tab_hbm.at[:, pl.ds(ch * 128, 128)], buf, sem)

    def wait_in(buf, sem):
      pltpu.make_async_copy(tab_hbm.at[:, pl.ds(0, 128)], buf, sem).wait()

    diag = [(iota + dlt) & 15 for dlt in range(16)]

    def transpose(vbuf, obuf):
      def rstep(rr, carry):
        r_idx = iota + rr * 16
        dst_r = iota * D + rr * 16 * D
        for d0 in (0, 16, 32, 48):
          vals = [plsc.load_gather(vbuf, [diag[dlt] + d0, r_idx])
                  for dlt in range(16)]
          for dlt in range(16):
            plsc.store_scatter(obuf, [dst_r + diag[dlt] + d0], vals[dlt])
        return carry

      lax.fori_loop(0, 8, rstep, 0)

    def flush(ch, buf, sem):
      pltpu.async_copy(buf, out_hbm.at[pl.ds(ch * 128 * D, 128 * D)], sem)

    def wait_out(buf, sem):
      pltpu.make_async_copy(buf, out_hbm.at[pl.ds(0, 128 * D)], sem).wait()

    for s in range(NSLOT):
      @pl.when(chunk_of(s) < NCH)
      def _(s=s):
        issue_in(chunk_of(s), vbufs[s], sis[s])

    def step(p, carry):
      for s in range(NSLOT):
        k = p * NSLOT + s
        ch = chunk_of(k)

        @pl.when(ch < NCH)
        def _(s=s, k=k, ch=ch):
          wait_in(vbufs[s], sis[s])

          @pl.when(k >= NSLOT)
          def _():
            wait_out(obufs[s], sos[s])

          transpose(vbufs[s], obufs[s])
          flush(ch, obufs[s], sos[s])
          nxt = ch + NW * NSLOT

          @pl.when(nxt < NCH)
          def _():
            issue_in(nxt, vbufs[s], sis[s])

      return carry

    lax.fori_loop(0, (NCH // NW + NSLOT) // NSLOT + 1, step, 0)
    for s in range(NSLOT):
      @pl.when(chunk_of(s) < NCH)
      def _(s=s):
        wait_out(obufs[s], sos[s])

    @pl.when(wid == 0)
    def _():
      rem = V - TAIL0
      pltpu.sync_copy(tail_hbm, v0)

      def dstep(d, carry):
        for k in range(rem // 16):
          val = v0[d, pl.ds(16 * k, 16)]
          plsc.store_scatter(o0, [iota64 + (k * 16 * D + d)], val)
        return carry

      lax.fori_loop(0, D, dstep, 0)
      pltpu.sync_copy(o0.at[pl.ds(0, rem * D)],
                      out_hbm.at[pl.ds(TAIL0 * D, rem * D)])

  return pl.kernel(
      body,
      out_type=jax.ShapeDtypeStruct((V * D,), jnp.float32),
      mesh=mesh,
      scratch_types=(
          [pltpu.VMEM((D, 128), jnp.float32)] * NSLOT
          + [pltpu.VMEM((128 * D,), jnp.float32)] * NSLOT
          + [pltpu.SemaphoreType.DMA] * (2 * NSLOT)
      ),
      compiler_params=pltpu.CompilerParams(use_tc_tiling_on_sc=True,
                                           needs_layout_passes=False),
  )(tableT, tail128)


def _sc_gather_sum(tokens_flat, table):
  mesh = plsc.VectorSubcoreMesh(core_axis_name="c", subcore_axis_name="s")

  def body(tokens_hbm, table_hbm, out_hbm, idx_v, buf_a, buf_b, out_v,
           sem_a, sem_b):
    wid = lax.axis_index("s") * NC + lax.axis_index("c")
    base = wid * BPW
    pltpu.sync_copy(tokens_hbm.at[pl.ds(base * L, BPW * L)], idx_v)

    def issue(bag, buf, sem):
      off = bag * L
      pltpu.async_copy(table_hbm.at[idx_v.at[pl.ds(off, C0)]],
                       buf.at[pl.ds(0, C0)], sem)
      pltpu.async_copy(table_hbm.at[idx_v.at[pl.ds(off + C0, C1)]],
                       buf.at[pl.ds(C0, C1)], sem)

    def wait(buf, sem):
      pltpu.make_async_copy(table_hbm.at[pl.ds(0, L)], buf, sem).wait()

    def accumulate(bag, buf):
      zeros = jnp.zeros((16,), jnp.float32)

      def step(i, accs):
        r = i * ROW_UNROLL
        new = list(accs)
        for dr in range(ROW_UNROLL):
          for j in range(4):
            new[j] = new[j] + buf[r + dr, pl.ds(16 * j, 16)]
        return tuple(new)

      accs = lax.fori_loop(0, L // ROW_UNROLL, step,
                           (zeros, zeros, zeros, zeros))
      for j in range(4):
        out_v[pl.ds(bag * D + 16 * j, 16)] = accs[j]

    issue(0, buf_a, sem_a)

    def pair(p, carry):
      bag = p * 2
      issue(bag + 1, buf_b, sem_b)
      wait(buf_a, sem_a)
      accumulate(bag, buf_a)

      @pl.when(bag + 2 < BPW)
      def _():
        issue(bag + 2, buf_a, sem_a)

      wait(buf_b, sem_b)
      accumulate(bag + 1, buf_b)
      return carry

    lax.fori_loop(0, BPW // 2, pair, 0)
    pltpu.sync_copy(out_v, out_hbm.at[pl.ds(base * D, BPW * D)])

  return pl.kernel(
      body,
      out_type=jax.ShapeDtypeStruct((B * D,), jnp.float32),
      mesh=mesh,
      scratch_types=[
          pltpu.VMEM((BPW * L,), jnp.int32),
          pltpu.VMEM((L, D), jnp.float32),
          pltpu.VMEM((L, D), jnp.float32),
          pltpu.VMEM((BPW * D,), jnp.float32),
          pltpu.SemaphoreType.DMA,
          pltpu.SemaphoreType.DMA,
      ],
      compiler_params=pltpu.CompilerParams(use_tc_tiling_on_sc=False),
  )(tokens_flat, table)


def _tc_head(tokens, sums, W1, b1, W2, b2):
  def body(tok_ref, sums_ref, w1_ref, b1_ref, w2_ref, b2_ref, out_ref):
    t = tok_ref[...]
    cnt = jnp.sum((t != 0).astype(jnp.float32), axis=1, keepdims=True)
    pooled = sums_ref[...] / jnp.maximum(cnt, 1.0)
    h = jnp.dot(pooled, w1_ref[...],
                preferred_element_type=jnp.float32) + b1_ref[...]
    h = 0.5 * h * (1.0 + lax.erf(h * (1.0 / math.sqrt(2.0))))
    out_ref[...] = jnp.dot(h, w2_ref[...],
                           preferred_element_type=jnp.float32) + b2_ref[...]

  grid = 8
  bb = B // grid
  return pl.pallas_call(
      body,
      out_shape=jax.ShapeDtypeStruct((B, O), jnp.float32),
      grid=(grid,),
      in_specs=[
          pl.BlockSpec((bb, L), lambda i: (i, 0)),
          pl.BlockSpec((bb, D), lambda i: (i, 0)),
          pl.BlockSpec((D, D), lambda i: (0, 0)),
          pl.BlockSpec((1, D), lambda i: (0, 0)),
          pl.BlockSpec((D, O), lambda i: (0, 0)),
          pl.BlockSpec((1, O), lambda i: (0, 0)),
      ],
      out_specs=pl.BlockSpec((bb, O), lambda i: (i, 0)),
  )(tokens, sums, W1, b1, W2, b2)


def kernel(tokens, table, W1, b1, W2, b2):
  tokens = tokens.astype(jnp.int32)
  tableT = table.T
  tail128 = jnp.pad(tableT[:, TAIL0:], ((0, 0), (0, 128 - (V - TAIL0))))
  table_lin = _sc_transpose(tableT, tail128).reshape(V, D)
  sums = _sc_gather_sum(tokens.reshape(-1), table_lin).reshape(B, D)
  return _tc_head(tokens, sums, W1, b1.reshape(1, D), W2, b2.reshape(1, O))


# speedup vs baseline: 3.3676x; 1.0973x over previous
import math

import jax
import jax.numpy as jnp
from jax import lax
from jax.experimental import pallas as pl
from jax.experimental.pallas import tpu as pltpu
from jax.experimental.pallas import tpu_sc as plsc

B, L, V, D, O = 4096, 200, 1000000, 64, 32
NC, NS = 2, 16
NW = NC * NS
BPW = B // NW
C0 = 128
C1 = L - C0
ROW_UNROLL = 8


NCH = 7812
NSLOT = 4
TAIL0 = NCH * 128


def _sc_transpose(tableT, tail128):
  mesh = plsc.VectorSubcoreMesh(core_axis_name="c", subcore_axis_name="s")

  def body(tab_hbm, tail_hbm, out_hbm, v0, v1, v2, v3,
           o0, o1, o2, o3, si0, si1, si2, si3, so0, so1, so2, so3):
    wid = lax.axis_index("s") * NC + lax.axis_index("c")
    vbufs = [v0, v1, v2, v3]
    obufs = [o0, o1, o2, o3]
    sis = [si0, si1, si2, si3]
    sos = [so0, so1, so2, so3]
    iota = lax.iota(jnp.int32, 16)
    iota64 = iota * D

    def chunk_of(k):
      return wid + NW * k

    def issue_in(ch, buf, sem):
      pltpu.async_copy(tab_hbm.at[:, pl.ds(ch * 128, 128)], buf, sem)

    def wait_in(buf, sem):
      pltpu.make_async_copy(tab_hbm.at[:, pl.ds(0, 128)], buf, sem).wait()

    diag = [(iota + dlt) & 15 for dlt in range(16)]

    def transpose(vbuf, obuf):
      def rstep(rr, carry):
        r_idx = iota + rr * 16
        dst_r = iota * D + rr * 16 * D
        for d0 in (0, 16, 32, 48):
          vals = [plsc.load_gather(vbuf, [diag[dlt] + d0, r_idx])
                  for dlt in range(16)]
          for dlt in range(16):
            plsc.store_scatter(obuf, [dst_r + diag[dlt] + d0], vals[dlt])
        return carry

      lax.fori_loop(0, 8, rstep, 0)

    def flush(ch, buf, sem):
      pltpu.async_copy(buf, out_hbm.at[pl.ds(ch * 128 * D, 128 * D)], sem)

    def wait_out(buf, sem):
      pltpu.make_async_copy(buf, out_hbm.at[pl.ds(0, 128 * D)], sem).wait()

    for s in range(NSLOT):
      @pl.when(chunk_of(s) < NCH)
      def _(s=s):
        issue_in(chunk_of(s), vbufs[s], sis[s])

    def step(p, carry):
      for s in range(NSLOT):
        k = p * NSLOT + s
        ch = chunk_of(k)

        @pl.when(ch < NCH)
        def _(s=s, k=k, ch=ch):
          wait_in(vbufs[s], sis[s])

          @pl.when(k >= NSLOT)
          def _():
            wait_out(obufs[s], sos[s])

          transpose(vbufs[s], obufs[s])
          flush(ch, obufs[s], sos[s])
          nxt = ch + NW * NSLOT

          @pl.when(nxt < NCH)
          def _():
            issue_in(nxt, vbufs[s], sis[s])

      return carry

    lax.fori_loop(0, (NCH // NW + NSLOT) // NSLOT + 1, step, 0)
    for s in range(NSLOT):
      @pl.when(chunk_of(s) < NCH)
      def _(s=s):
        wait_out(obufs[s], sos[s])

    @pl.when(wid == 0)
    def _():
      rem = V - TAIL0
      pltpu.sync_copy(tail_hbm, v0)

      def dstep(d, carry):
        for k in range(rem // 16):
          val = v0[d, pl.ds(16 * k, 16)]
          plsc.store_scatter(o0, [iota64 + (k * 16 * D + d)], val)
        return carry

      lax.fori_loop(0, D, dstep, 0)
      pltpu.sync_copy(o0.at[pl.ds(0, rem * D)],
                      out_hbm.at[pl.ds(TAIL0 * D, rem * D)])

  return pl.kernel(
      body,
      out_type=jax.ShapeDtypeStruct((V * D,), jnp.float32),
      mesh=mesh,
      scratch_types=(
          [pltpu.VMEM((D, 128), jnp.float32)] * NSLOT
          + [pltpu.VMEM((128 * D,), jnp.float32)] * NSLOT
          + [pltpu.SemaphoreType.DMA] * (2 * NSLOT)
      ),
      compiler_params=pltpu.CompilerParams(use_tc_tiling_on_sc=True,
                                           needs_layout_passes=False),
  )(tableT, tail128)


def _sc_gather_sum(tokens_flat, table):
  mesh = plsc.VectorSubcoreMesh(core_axis_name="c", subcore_axis_name="s")

  def body(tokens_hbm, table_hbm, out_hbm, idx_v, b0, b1, b2, b3, out_v,
           s0, s1, s2, s3):
    wid = lax.axis_index("s") * NC + lax.axis_index("c")
    base = wid * BPW
    bufs = [b0, b1, b2, b3]
    sems = [s0, s1, s2, s3]
    pltpu.sync_copy(tokens_hbm.at[pl.ds(base * L, BPW * L)], idx_v)

    def issue(bag, buf, sem):
      off = bag * L
      pltpu.async_copy(table_hbm.at[idx_v.at[pl.ds(off, C0)]],
                       buf.at[pl.ds(0, C0)], sem)
      pltpu.async_copy(table_hbm.at[idx_v.at[pl.ds(off + C0, C1)]],
                       buf.at[pl.ds(C0, C1)], sem)

    def wait(buf, sem):
      pltpu.make_async_copy(table_hbm.at[pl.ds(0, L)], buf, sem).wait()

    def accumulate(bag, buf):
      zeros = jnp.zeros((16,), jnp.float32)

      def step(i, accs):
        r = i * ROW_UNROLL
        new = list(accs)
        for dr in range(ROW_UNROLL):
          for j in range(4):
            new[j] = new[j] + buf[r + dr, pl.ds(16 * j, 16)]
        return tuple(new)

      accs = lax.fori_loop(0, L // ROW_UNROLL, step,
                           (zeros, zeros, zeros, zeros))
      for j in range(4):
        out_v[pl.ds(bag * D + 16 * j, 16)] = accs[j]

    for s in range(4):
      issue(s, bufs[s], sems[s])

    def quad(p, carry):
      for s in range(4):
        bag = p * 4 + s
        wait(bufs[s], sems[s])
        accumulate(bag, bufs[s])

        @pl.when(bag + 4 < BPW)
        def _(s=s, bag=bag):
          issue(bag + 4, bufs[s], sems[s])

      return carry

    lax.fori_loop(0, BPW // 4, quad, 0)
    pltpu.sync_copy(out_v, out_hbm.at[pl.ds(base * D, BPW * D)])

  return pl.kernel(
      body,
      out_type=jax.ShapeDtypeStruct((B * D,), jnp.float32),
      mesh=mesh,
      scratch_types=(
          [pltpu.VMEM((BPW * L,), jnp.int32)]
          + [pltpu.VMEM((L, D), jnp.float32)] * 4
          + [pltpu.VMEM((BPW * D,), jnp.float32)]
          + [pltpu.SemaphoreType.DMA] * 4
      ),
      compiler_params=pltpu.CompilerParams(use_tc_tiling_on_sc=False),
  )(tokens_flat, table)


def _tc_head(tokens, sums, W1, b1, W2, b2):
  def body(tok_ref, sums_ref, w1_ref, b1_ref, w2_ref, b2_ref, out_ref):
    t = tok_ref[...]
    cnt = jnp.sum((t != 0).astype(jnp.float32), axis=1, keepdims=True)
    pooled = sums_ref[...] / jnp.maximum(cnt, 1.0)
    h = jnp.dot(pooled, w1_ref[...],
                preferred_element_type=jnp.float32) + b1_ref[...]
    h = 0.5 * h * (1.0 + lax.erf(h * (1.0 / math.sqrt(2.0))))
    out_ref[...] = jnp.dot(h, w2_ref[...],
                           preferred_element_type=jnp.float32) + b2_ref[...]

  grid = 8
  bb = B // grid
  return pl.pallas_call(
      body,
      out_shape=jax.ShapeDtypeStruct((B, O), jnp.float32),
      grid=(grid,),
      in_specs=[
          pl.BlockSpec((bb, L), lambda i: (i, 0)),
          pl.BlockSpec((bb, D), lambda i: (i, 0)),
          pl.BlockSpec((D, D), lambda i: (0, 0)),
          pl.BlockSpec((1, D), lambda i: (0, 0)),
          pl.BlockSpec((D, O), lambda i: (0, 0)),
          pl.BlockSpec((1, O), lambda i: (0, 0)),
      ],
      out_specs=pl.BlockSpec((bb, O), lambda i: (i, 0)),
  )(tokens, sums, W1, b1, W2, b2)


def kernel(tokens, table, W1, b1, W2, b2):
  tokens = tokens.astype(jnp.int32)
  tableT = table.T
  tail128 = jnp.pad(tableT[:, TAIL0:], ((0, 0), (0, 128 - (V - TAIL0))))
  table_lin = _sc_transpose(tableT, tail128).reshape(V, D)
  sums = _sc_gather_sum(tokens.reshape(-1), table_lin).reshape(B, D)
  return _tc_head(tokens, sums, W1, b1.reshape(1, D), W2, b2.reshape(1, O))
